# baseline (speedup 1.0000x reference)
import functools

import jax
import jax.numpy as jnp
from jax import lax
from jax.experimental import pallas as pl
from jax.experimental.pallas import tpu as pltpu
from jax.experimental.pallas import tpu_sc as plsc

NN = 10000
EE = 320000
DD = 128
RR = 4
NP_ = 10240
NC, NS = 2, 16
NW = NC * NS
CH = 80
NBUF = 2
EW = CH * 128
NPA = 10112
RPT = NPA // NS
ECH = 8
GG = 128
SEQ = 5
VV = 5000
VP = 5120
RB = 256
NB = NP_ // RB
LAYER_T = [2, 2, 1, 2, 1]
RES_MAP = {2: [0], 4: [0, 2]}


def _embed_gather_body(table, eidx, out, idx_v, buf, sem):
    c = lax.axis_index("c")
    s = lax.axis_index("s")
    wid = s * NC + c
    pltpu.sync_copy(eidx.at[wid], idx_v)
    for j in range(ECH):
        pltpu.async_copy(table.at[idx_v.at[j]], buf, sem).wait()
        pltpu.sync_copy(buf, out.at[pl.ds(wid * (ECH * 128) + j * 128, 128)])


@functools.cache
def _sc_mesh():
    return plsc.VectorSubcoreMesh(core_axis_name="c", subcore_axis_name="s")


@functools.cache
def _embed_gather_kernel():
    return pl.kernel(
        _embed_gather_body,
        out_type=jax.ShapeDtypeStruct((NW * ECH * 128, DD), jnp.float32),
        mesh=_sc_mesh(),
        scratch_types=[
            pltpu.VMEM((ECH, 128), jnp.int32),
            pltpu.VMEM((128, DD), jnp.float32),
            pltpu.SemaphoreType.DMA,
        ],
    )


def _embed_gather(table, eidx_w):
    return _embed_gather_kernel()(table, eidx_w)


def _unpack_row(packed, j, stage, b):
    half = (j % 2) * 64
    for k in range(4):
        w = packed[j // 2, pl.ds(half + k * 16, 16)]
        stage[b, pl.ds(k * 16, 16)] = (w & 0xFFFF).astype(jnp.int32)
        stage[b, pl.ds(64 + k * 16, 16)] = (w >> 16).astype(jnp.int32)


def _edge_gs_body(hr, gidx, dstl, zeros, out, shared, gbuf, gidx_v, dst_v,
                  zsem, gsem):
    c = lax.axis_index("c")
    s = lax.axis_index("s")
    wid = s * NC + c
    pltpu.sync_copy(gidx.at[wid], gidx_v)
    pltpu.sync_copy(dstl.at[wid], dst_v)
    zcopy = pltpu.async_copy(
        zeros.at[pl.ds(s * RPT, RPT)], shared.at[pl.ds(s * RPT, RPT)], zsem)
    zcopy.wait()
    plsc.subcore_barrier()

    def body(j, carry):
        pltpu.async_copy(hr.at[gidx_v.at[j]], gbuf, gsem).wait()
        pltpu.sync_copy(gbuf, shared.at[dst_v.at[j]], add=True)
        return carry

    lax.fori_loop(0, CH, body, 0)
    plsc.subcore_barrier()
    pltpu.sync_copy(
        shared.at[pl.ds(s * RPT, RPT)], out.at[pl.ds(c * NP_ + s * RPT, RPT)]
    )


@functools.cache
def _edge_gs_kernel():
    return pl.kernel(
        _edge_gs_body,
        out_type=jax.ShapeDtypeStruct((NC * NP_, DD), jnp.float32),
        mesh=_sc_mesh(),
        scratch_types=[
            pltpu.VMEM_SHARED((NPA, DD), jnp.float32),
            pltpu.VMEM((128, DD), jnp.float32),
            pltpu.VMEM((CH, 128), jnp.int32),
            pltpu.VMEM((CH, 128), jnp.int32),
        ] + [pltpu.SemaphoreType.DMA] * 2,
    )


def _edge_gs(hr_flat, gidx_p, dst_p, zeros_np):
    return _edge_gs_kernel()(hr_flat, gidx_p, dst_p, zeros_np)


def _a0_body(e3, w, h0_ref, hr_ref):
    h0 = e3[0] + e3[1] + e3[2]
    h0_ref[...] = h0
    for r in range(RR):
        hr_ref[r] = jnp.dot(h0, w[r], preferred_element_type=jnp.float32)


_a0 = pl.pallas_call(
    _a0_body,
    grid=(NB,),
    in_specs=[
        pl.BlockSpec((3, RB, DD), lambda i: (0, i, 0)),
        pl.BlockSpec((RR, DD, DD), lambda i: (0, 0, 0)),
    ],
    out_specs=[
        pl.BlockSpec((RB, DD), lambda i: (i, 0)),
        pl.BlockSpec((RR, RB, DD), lambda i: (0, i, 0)),
    ],
    out_shape=[
        jax.ShapeDtypeStruct((NP_, DD), jnp.float32),
        jax.ShapeDtypeStruct((RR, NP_, DD), jnp.float32),
    ],
)


@functools.cache
def _make_step(nres, with_hr):
    def body(*refs):
        h = refs[0][...]
        agg2 = refs[1]
        res = [refs[2 + i][...] for i in range(nres)]
        k = 2 + nres
        wih, whh, bih, bhh = refs[k], refs[k + 1], refs[k + 2], refs[k + 3]
        hnew_ref = refs[k + 4 + (1 if with_hr else 0)]
        agg = agg2[0] + agg2[1]
        xin = jnp.concatenate([agg] + res, axis=1) if nres else agg
        gi = lax.dot_general(
            xin, wih[...], (((1,), (1,)), ((), ())),
            preferred_element_type=jnp.float32,
        ) + bih[...]
        gh = lax.dot_general(
            h, whh[...], (((1,), (1,)), ((), ())),
            preferred_element_type=jnp.float32,
        ) + bhh[...]
        r = jax.nn.sigmoid(gi[:, :DD] + gh[:, :DD])
        z = jax.nn.sigmoid(gi[:, DD:2 * DD] + gh[:, DD:2 * DD])
        n = jnp.tanh(gi[:, 2 * DD:] + r * gh[:, 2 * DD:])
        hn = (1.0 - z) * n + z * h
        hnew_ref[...] = hn
        if with_hr:
            w = refs[k + 4]
            hr_ref = refs[k + 6]
            for rr in range(RR):
                hr_ref[rr] = jnp.dot(hn, w[rr], preferred_element_type=jnp.float32)

    in_dim = DD * (1 + nres)
    in_specs = [
        pl.BlockSpec((RB, DD), lambda i: (i, 0)),
        pl.BlockSpec((NC, RB, DD), lambda i: (0, i, 0)),
    ]
    for _ in range(nres):
        in_specs.append(pl.BlockSpec((RB, DD), lambda i: (i, 0)))
    in_specs += [
        pl.BlockSpec((3 * DD, in_dim), lambda i: (0, 0)),
        pl.BlockSpec((3 * DD, DD), lambda i: (0, 0)),
        pl.BlockSpec((1, 3 * DD), lambda i: (0, 0)),
        pl.BlockSpec((1, 3 * DD), lambda i: (0, 0)),
    ]
    out_specs = [pl.BlockSpec((RB, DD), lambda i: (i, 0))]
    out_shape = [jax.ShapeDtypeStruct((NP_, DD), jnp.float32)]
    if with_hr:
        in_specs.append(pl.BlockSpec((RR, DD, DD), lambda i: (0, 0, 0)))
        out_specs.append(pl.BlockSpec((RR, RB, DD), lambda i: (0, i, 0)))
        out_shape.append(jax.ShapeDtypeStruct((RR, NP_, DD), jnp.float32))
    return pl.pallas_call(
        body,
        grid=(NB,),
        in_specs=in_specs,
        out_specs=out_specs,
        out_shape=out_shape,
    )


def _c1_body(hf, h0, bt, clw, clb, crw, crb, g_ref):
    i = pl.program_id(0)
    hx = jnp.concatenate([hf[...], h0[...]], axis=1)
    a = jax.nn.sigmoid(
        lax.dot_general(hx, clw[...], (((1,), (1,)), ((), ())),
                        preferred_element_type=jnp.float32) + clb[...])
    b = jnp.tanh(
        lax.dot_general(hx, crw[...], (((1,), (1,)), ((), ())),
                        preferred_element_type=jnp.float32) + crb[...])
    node_out = a * b
    gids = lax.broadcasted_iota(jnp.int32, (RB, GG), 1)
    onehot = (bt[...] == gids).astype(jnp.float32)
    pool = lax.dot_general(onehot, node_out, (((0,), (0,)), ((), ())),
                           preferred_element_type=jnp.float32)

    @pl.when(i == 0)
    def _():
        g_ref[...] = jnp.zeros_like(g_ref)

    g_ref[...] += pool


_c1 = pl.pallas_call(
    _c1_body,
    grid=(NB,),
    in_specs=[
        pl.BlockSpec((RB, DD), lambda i: (i, 0)),
        pl.BlockSpec((RB, DD), lambda i: (i, 0)),
        pl.BlockSpec((RB, 1), lambda i: (i, 0)),
        pl.BlockSpec((DD, 2 * DD), lambda i: (0, 0)),
        pl.BlockSpec((1, DD), lambda i: (0, 0)),
        pl.BlockSpec((DD, 2 * DD), lambda i: (0, 0)),
        pl.BlockSpec((1, DD), lambda i: (0, 0)),
    ],
    out_specs=pl.BlockSpec((GG, DD), lambda i: (0, 0)),
    out_shape=jax.ShapeDtypeStruct((GG, DD), jnp.float32),
)


def _c2_body(g, pw, pb, out):
    out[0] = lax.dot_general(g[...], pw[0], (((1,), (1,)), ((), ())),
                             preferred_element_type=jnp.float32) + pb[0]


_c2 = pl.pallas_call(
    _c2_body,
    grid=(SEQ,),
    in_specs=[
        pl.BlockSpec((GG, DD), lambda s: (0, 0)),
        pl.BlockSpec((1, VP, DD), lambda s: (s, 0, 0)),
        pl.BlockSpec((1, 1, VP), lambda s: (s, 0, 0)),
    ],
    out_specs=pl.BlockSpec((1, GG, VP), lambda s: (s, 0, 0)),
    out_shape=jax.ShapeDtypeStruct((SEQ, GG, VP), jnp.float32),
)


def kernel(x, edge_index, node_depth, batch, edge_attr, params):
    i32 = jnp.int32
    f32 = jnp.float32
    x = x.astype(i32)
    src = edge_index[0].astype(i32)
    dst = edge_index[1].astype(i32)
    rel = edge_attr.astype(i32)

    table = jnp.concatenate(
        [params['type_emb'], params['attr_emb'], params['depth_emb']], axis=0)
    eidx = jnp.concatenate(
        [x[:, 0], 100 + x[:, 1], 1100 + node_depth.reshape(-1).astype(i32)])
    eidx3 = jnp.zeros((3, NP_), i32).at[:, :NN].set(eidx.reshape(3, NN))
    per_w = 3 * NP_ // NW
    eidx_w = (jnp.zeros((NW, ECH * 128), i32)
              .at[:, :per_w].set(eidx3.reshape(NW, per_w))
              .reshape(NW, ECH, 128))
    eout = _embed_gather(table, eidx_w)
    e3 = eout.reshape(NW, ECH * 128, DD)[:, :per_w].reshape(3, NP_, DD)

    gidx = rel * NP_ + src
    gidx_p = jnp.zeros((NW * EW,), i32).at[:EE].set(gidx).reshape(NW, CH, 128)
    dst_p = jnp.full((NW * EW,), NN, i32).at[:EE].set(dst).reshape(NW, CH, 128)
    zeros_np = jnp.zeros((NP_, DD), f32)

    steps = [l for l, T in enumerate(LAYER_T) for _ in range(T)]
    h, hr = _a0(e3, params['edge_w_0'])
    h0 = h
    states = [h0]
    for k, l in enumerate(steps):
        agg2 = _edge_gs(hr.reshape(RR * NP_, DD), gidx_p, dst_p, zeros_np)
        agg2 = agg2.reshape(NC, NP_, DD)
        res = [states[i] for i in RES_MAP.get(l, [])]
        last = k + 1 == len(steps)
        step_fn = _make_step(len(res), not last)
        args = [h, agg2] + res + [
            params['gru_wih_%d' % l],
            params['gru_whh_%d' % l],
            params['gru_bih_%d' % l].reshape(1, 3 * DD),
            params['gru_bhh_%d' % l].reshape(1, 3 * DD),
        ]
        if last:
            (h,) = step_fn(*args)
        else:
            nl = steps[k + 1]
            h, hr = step_fn(*(args + [params['edge_w_%d' % nl]]))
        if last or steps[k + 1] != l:
            states.append(h)

    batch_p = jnp.full((NP_, 1), GG, i32).at[:NN, 0].set(batch.astype(i32))
    g = _c1(states[-1], h0, batch_p,
            params['cl_w'], params['cl_b'].reshape(1, DD),
            params['cr_w'], params['cr_b'].reshape(1, DD))
    pw = jnp.zeros((SEQ, VP, DD), f32).at[:, :VV].set(params['pred_w'])
    pb = jnp.zeros((SEQ, 1, VP), f32).at[:, 0, :VV].set(params['pred_b'])
    preds = _c2(g, pw, pb)
    return preds[:, :, :VV]


# speedup vs baseline: 1.4183x; 1.4183x over previous
import functools

import jax
import jax.numpy as jnp
from jax import lax
from jax.experimental import pallas as pl
from jax.experimental.pallas import tpu as pltpu
from jax.experimental.pallas import tpu_sc as plsc

NN = 10000
EE = 320000
DD = 128
RR = 4
NP_ = 10240
NC, NS = 2, 16
NW = NC * NS
CH = 79
EW = CH * 128
NPA = NP_
RPT = NPA // NS
ECH = 8
GG = 128
SEQ = 5
VV = 5000
VP = 5120
RB = 256
NB = NP_ // RB
LAYER_T = [2, 2, 1, 2, 1]
RES_MAP = {2: [0], 4: [0, 2]}


def _embed_gather_body(table, eidx, out, idx_v, buf, sem):
    c = lax.axis_index("c")
    s = lax.axis_index("s")
    wid = s * NC + c
    pltpu.sync_copy(eidx.at[wid], idx_v)
    for j in range(ECH):
        pltpu.async_copy(table.at[idx_v.at[j]], buf, sem).wait()
        pltpu.sync_copy(buf, out.at[pl.ds(wid * (ECH * 128) + j * 128, 128)])


@functools.cache
def _sc_mesh():
    return plsc.VectorSubcoreMesh(core_axis_name="c", subcore_axis_name="s")


@functools.cache
def _embed_gather_kernel():
    return pl.kernel(
        _embed_gather_body,
        out_type=jax.ShapeDtypeStruct((NW * ECH * 128, DD), jnp.float32),
        mesh=_sc_mesh(),
        scratch_types=[
            pltpu.VMEM((ECH, 128), jnp.int32),
            pltpu.VMEM((128, DD), jnp.float32),
            pltpu.SemaphoreType.DMA,
        ],
    )


def _embed_gather(table, eidx_w):
    return _embed_gather_kernel()(table, eidx_w)


def _unpack_row(packed, j, stage, b):
    half = (j % 2) * 64
    for k in range(4):
        w = packed[j // 2, pl.ds(half + k * 16, 16)]
        stage[b, pl.ds(k * 16, 16)] = (w & 0xFFFF).astype(jnp.int32)
        stage[b, pl.ds(64 + k * 16, 16)] = (w >> 16).astype(jnp.int32)


def _edge_gs_body(hr, gidx, dstl, zeros, out, shared, gbuf, gidx_v, dst_v,
                  gsem):
    c = lax.axis_index("c")
    s = lax.axis_index("s")
    wid = s * NC + c
    pltpu.sync_copy(zeros.at[pl.ds(s * RPT, RPT)], shared.at[pl.ds(s * RPT, RPT)])
    pltpu.sync_copy(gidx.at[wid], gidx_v)
    pltpu.sync_copy(dstl.at[wid], dst_v)
    plsc.subcore_barrier()

    def body(j, carry):
        pltpu.async_copy(hr.at[gidx_v.at[j]], gbuf, gsem).wait()
        pltpu.sync_copy(gbuf, shared.at[dst_v.at[j]], add=True)
        return carry

    lax.fori_loop(0, CH, body, 0)
    plsc.subcore_barrier()
    pltpu.sync_copy(
        shared.at[pl.ds(s * RPT, RPT)], out.at[pl.ds(c * NP_ + s * RPT, RPT)]
    )


@functools.cache
def _edge_gs_kernel():
    return pl.kernel(
        _edge_gs_body,
        out_type=jax.ShapeDtypeStruct((NC * NP_, DD), jnp.float32),
        mesh=_sc_mesh(),
        scratch_types=[
            pltpu.VMEM_SHARED((NPA, DD), jnp.float32),
            pltpu.VMEM((128, DD), jnp.float32),
            pltpu.VMEM((CH, 128), jnp.int32),
            pltpu.VMEM((CH, 128), jnp.int32),
            pltpu.SemaphoreType.DMA,
        ],
    )


def _edge_gs(hr_flat, gidx_p, dst_p, zeros_np):
    return _edge_gs_kernel()(hr_flat, gidx_p, dst_p, zeros_np)


def _a0_body(e3, w, h0_ref, hr_ref):
    h0 = e3[0] + e3[1] + e3[2]
    h0_ref[...] = h0
    for r in range(RR):
        hr_ref[r] = jnp.dot(h0, w[r], preferred_element_type=jnp.float32)


_a0 = pl.pallas_call(
    _a0_body,
    grid=(NB,),
    in_specs=[
        pl.BlockSpec((3, RB, DD), lambda i: (0, i, 0)),
        pl.BlockSpec((RR, DD, DD), lambda i: (0, 0, 0)),
    ],
    out_specs=[
        pl.BlockSpec((RB, DD), lambda i: (i, 0)),
        pl.BlockSpec((RR, RB, DD), lambda i: (0, i, 0)),
    ],
    out_shape=[
        jax.ShapeDtypeStruct((NP_, DD), jnp.float32),
        jax.ShapeDtypeStruct((RR, NP_, DD), jnp.float32),
    ],
)


@functools.cache
def _make_step(nres, with_hr):
    def body(*refs):
        h = refs[0][...]
        agg2 = refs[1]
        res = [refs[2 + i][...] for i in range(nres)]
        k = 2 + nres
        wih, whh, bih, bhh = refs[k], refs[k + 1], refs[k + 2], refs[k + 3]
        hnew_ref = refs[k + 4 + (1 if with_hr else 0)]
        agg = agg2[0] + agg2[1]
        xin = jnp.concatenate([agg] + res, axis=1) if nres else agg
        gi = lax.dot_general(
            xin, wih[...], (((1,), (1,)), ((), ())),
            preferred_element_type=jnp.float32,
        ) + bih[...]
        gh = lax.dot_general(
            h, whh[...], (((1,), (1,)), ((), ())),
            preferred_element_type=jnp.float32,
        ) + bhh[...]
        r = jax.nn.sigmoid(gi[:, :DD] + gh[:, :DD])
        z = jax.nn.sigmoid(gi[:, DD:2 * DD] + gh[:, DD:2 * DD])
        n = jnp.tanh(gi[:, 2 * DD:] + r * gh[:, 2 * DD:])
        hn = (1.0 - z) * n + z * h
        hnew_ref[...] = hn
        if with_hr:
            w = refs[k + 4]
            hr_ref = refs[k + 6]
            for rr in range(RR):
                hr_ref[rr] = jnp.dot(hn, w[rr], preferred_element_type=jnp.float32)

    in_dim = DD * (1 + nres)
    in_specs = [
        pl.BlockSpec((RB, DD), lambda i: (i, 0)),
        pl.BlockSpec((NC, RB, DD), lambda i: (0, i, 0)),
    ]
    for _ in range(nres):
        in_specs.append(pl.BlockSpec((RB, DD), lambda i: (i, 0)))
    in_specs += [
        pl.BlockSpec((3 * DD, in_dim), lambda i: (0, 0)),
        pl.BlockSpec((3 * DD, DD), lambda i: (0, 0)),
        pl.BlockSpec((1, 3 * DD), lambda i: (0, 0)),
        pl.BlockSpec((1, 3 * DD), lambda i: (0, 0)),
    ]
    out_specs = [pl.BlockSpec((RB, DD), lambda i: (i, 0))]
    out_shape = [jax.ShapeDtypeStruct((NP_, DD), jnp.float32)]
    if with_hr:
        in_specs.append(pl.BlockSpec((RR, DD, DD), lambda i: (0, 0, 0)))
        out_specs.append(pl.BlockSpec((RR, RB, DD), lambda i: (0, i, 0)))
        out_shape.append(jax.ShapeDtypeStruct((RR, NP_, DD), jnp.float32))
    return pl.pallas_call(
        body,
        grid=(NB,),
        in_specs=in_specs,
        out_specs=out_specs,
        out_shape=out_shape,
    )


def _c1_body(hf, h0, bt, clw, clb, crw, crb, g_ref):
    i = pl.program_id(0)
    hx = jnp.concatenate([hf[...], h0[...]], axis=1)
    a = jax.nn.sigmoid(
        lax.dot_general(hx, clw[...], (((1,), (1,)), ((), ())),
                        preferred_element_type=jnp.float32) + clb[...])
    b = jnp.tanh(
        lax.dot_general(hx, crw[...], (((1,), (1,)), ((), ())),
                        preferred_element_type=jnp.float32) + crb[...])
    node_out = a * b
    gids = lax.broadcasted_iota(jnp.int32, (RB, GG), 1)
    onehot = (bt[...] == gids).astype(jnp.float32)
    pool = lax.dot_general(onehot, node_out, (((0,), (0,)), ((), ())),
                           preferred_element_type=jnp.float32)

    @pl.when(i == 0)
    def _():
        g_ref[...] = jnp.zeros_like(g_ref)

    g_ref[...] += pool


_c1 = pl.pallas_call(
    _c1_body,
    grid=(NB,),
    in_specs=[
        pl.BlockSpec((RB, DD), lambda i: (i, 0)),
        pl.BlockSpec((RB, DD), lambda i: (i, 0)),
        pl.BlockSpec((RB, 1), lambda i: (i, 0)),
        pl.BlockSpec((DD, 2 * DD), lambda i: (0, 0)),
        pl.BlockSpec((1, DD), lambda i: (0, 0)),
        pl.BlockSpec((DD, 2 * DD), lambda i: (0, 0)),
        pl.BlockSpec((1, DD), lambda i: (0, 0)),
    ],
    out_specs=pl.BlockSpec((GG, DD), lambda i: (0, 0)),
    out_shape=jax.ShapeDtypeStruct((GG, DD), jnp.float32),
)


def _c2_body(g, pw, pb, out):
    out[0] = lax.dot_general(g[...], pw[0], (((1,), (1,)), ((), ())),
                             preferred_element_type=jnp.float32) + pb[0]


_c2 = pl.pallas_call(
    _c2_body,
    grid=(SEQ,),
    in_specs=[
        pl.BlockSpec((GG, DD), lambda s: (0, 0)),
        pl.BlockSpec((1, VP, DD), lambda s: (s, 0, 0)),
        pl.BlockSpec((1, 1, VP), lambda s: (s, 0, 0)),
    ],
    out_specs=pl.BlockSpec((1, GG, VP), lambda s: (s, 0, 0)),
    out_shape=jax.ShapeDtypeStruct((SEQ, GG, VP), jnp.float32),
)


def kernel(x, edge_index, node_depth, batch, edge_attr, params):
    i32 = jnp.int32
    f32 = jnp.float32
    x = x.astype(i32)
    src = edge_index[0].astype(i32)
    dst = edge_index[1].astype(i32)
    rel = edge_attr.astype(i32)

    table = jnp.concatenate(
        [params['type_emb'], params['attr_emb'], params['depth_emb']], axis=0)
    eidx = jnp.concatenate(
        [x[:, 0], 100 + x[:, 1], 1100 + node_depth.reshape(-1).astype(i32)])
    eidx3 = jnp.zeros((3, NP_), i32).at[:, :NN].set(eidx.reshape(3, NN))
    per_w = 3 * NP_ // NW
    eidx_w = (jnp.zeros((NW, ECH * 128), i32)
              .at[:, :per_w].set(eidx3.reshape(NW, per_w))
              .reshape(NW, ECH, 128))
    eout = _embed_gather(table, eidx_w)
    e3 = eout.reshape(NW, ECH * 128, DD)[:, :per_w].reshape(3, NP_, DD)

    gidx = rel * NP_ + src
    gidx_p = jnp.zeros((NW * EW,), i32).at[:EE].set(gidx).reshape(NW, CH, 128)
    dst_p = jnp.full((NW * EW,), NN, i32).at[:EE].set(dst).reshape(NW, CH, 128)
    zeros_np = jnp.zeros((NP_, DD), f32)

    steps = [l for l, T in enumerate(LAYER_T) for _ in range(T)]
    h, hr = _a0(e3, params['edge_w_0'])
    h0 = h
    states = [h0]
    for k, l in enumerate(steps):
        agg2 = _edge_gs(hr.reshape(RR * NP_, DD), gidx_p, dst_p, zeros_np)
        agg2 = agg2.reshape(NC, NP_, DD)
        res = [states[i] for i in RES_MAP.get(l, [])]
        last = k + 1 == len(steps)
        step_fn = _make_step(len(res), not last)
        args = [h, agg2] + res + [
            params['gru_wih_%d' % l],
            params['gru_whh_%d' % l],
            params['gru_bih_%d' % l].reshape(1, 3 * DD),
            params['gru_bhh_%d' % l].reshape(1, 3 * DD),
        ]
        if last:
            (h,) = step_fn(*args)
        else:
            nl = steps[k + 1]
            h, hr = step_fn(*(args + [params['edge_w_%d' % nl]]))
        if last or steps[k + 1] != l:
            states.append(h)

    batch_p = jnp.full((NP_, 1), GG, i32).at[:NN, 0].set(batch.astype(i32))
    g = _c1(states[-1], h0, batch_p,
            params['cl_w'], params['cl_b'].reshape(1, DD),
            params['cr_w'], params['cr_b'].reshape(1, DD))
    pw = jnp.zeros((SEQ, VP, DD), f32).at[:, :VV].set(params['pred_w'])
    pb = jnp.zeros((SEQ, 1, VP), f32).at[:, 0, :VV].set(params['pred_b'])
    preds = _c2(g, pw, pb)
    return preds[:, :, :VV]


# speedup vs baseline: 1.5357x; 1.0827x over previous
import functools

import jax
import jax.numpy as jnp
from jax import lax
from jax.experimental import pallas as pl
from jax.experimental.pallas import tpu as pltpu
from jax.experimental.pallas import tpu_sc as plsc

NN = 10000
EE = 320000
DD = 128
RR = 4
NP_ = 10240
NC, NS = 2, 16
NW = NC * NS
CH = 79
CH0, CH1 = 99, 59
EW = CH * 128
NPA = NP_
RPT = NPA // NS
ECH = 8
GG = 128
SEQ = 5
VV = 5000
VP = 5120
RB = 256
NB = NP_ // RB
LAYER_T = [2, 2, 1, 2, 1]
RES_MAP = {2: [0], 4: [0, 2]}


def _embed_gather_body(table, eidx, out, idx_v, buf, sem):
    c = lax.axis_index("c")
    s = lax.axis_index("s")
    wid = s * NC + c
    pltpu.sync_copy(eidx.at[wid], idx_v)
    for j in range(ECH):
        pltpu.async_copy(table.at[idx_v.at[j]], buf, sem).wait()
        pltpu.sync_copy(buf, out.at[pl.ds(wid * (ECH * 128) + j * 128, 128)])


@functools.cache
def _sc_mesh():
    return plsc.VectorSubcoreMesh(core_axis_name="c", subcore_axis_name="s")


@functools.cache
def _embed_gather_kernel():
    return pl.kernel(
        _embed_gather_body,
        out_type=jax.ShapeDtypeStruct((NW * ECH * 128, DD), jnp.float32),
        mesh=_sc_mesh(),
        scratch_types=[
            pltpu.VMEM((ECH, 128), jnp.int32),
            pltpu.VMEM((128, DD), jnp.float32),
            pltpu.SemaphoreType.DMA,
        ],
    )


def _embed_gather(table, eidx_w):
    return _embed_gather_kernel()(table, eidx_w)


def _unpack_row(packed, j, stage, b):
    half = (j % 2) * 64
    for k in range(4):
        w = packed[j // 2, pl.ds(half + k * 16, 16)]
        stage[b, pl.ds(k * 16, 16)] = (w & 0xFFFF).astype(jnp.int32)
        stage[b, pl.ds(64 + k * 16, 16)] = (w >> 16).astype(jnp.int32)


def _edge_gs_body(hr, gidx, dstl, zeros, out, shared, gbuf, gidx_v, dst_v,
                  gsem):
    c = lax.axis_index("c")
    s = lax.axis_index("s")
    widx = c * NS + s
    pltpu.sync_copy(zeros.at[pl.ds(s * RPT, RPT)], shared.at[pl.ds(s * RPT, RPT)])
    pltpu.sync_copy(gidx.at[widx], gidx_v)
    pltpu.sync_copy(dstl.at[widx], dst_v)
    plsc.subcore_barrier()

    def body(j, carry):
        pltpu.async_copy(hr.at[gidx_v.at[j]], gbuf, gsem).wait()
        pltpu.sync_copy(gbuf, shared.at[dst_v.at[j]], add=True)
        return carry

    lax.fori_loop(0, jnp.where(c == 0, CH0, CH1), body, 0)
    plsc.subcore_barrier()
    pltpu.sync_copy(
        shared.at[pl.ds(s * RPT, RPT)], out.at[pl.ds(c * NP_ + s * RPT, RPT)]
    )


@functools.cache
def _edge_gs_kernel():
    return pl.kernel(
        _edge_gs_body,
        out_type=jax.ShapeDtypeStruct((NC * NP_, DD), jnp.float32),
        mesh=_sc_mesh(),
        scratch_types=[
            pltpu.VMEM_SHARED((NPA, DD), jnp.float32),
            pltpu.VMEM((128, DD), jnp.float32),
            pltpu.VMEM((CH0, 128), jnp.int32),
            pltpu.VMEM((CH0, 128), jnp.int32),
            pltpu.SemaphoreType.DMA,
        ],
    )


def _edge_gs(hr_flat, gidx_p, dst_p, zeros_np):
    return _edge_gs_kernel()(hr_flat, gidx_p, dst_p, zeros_np)


def _a0_body(e3, w, h0_ref, hr_ref):
    h0 = e3[0] + e3[1] + e3[2]
    h0_ref[...] = h0
    for r in range(RR):
        hr_ref[r] = jnp.dot(h0, w[r], preferred_element_type=jnp.float32)


_a0 = pl.pallas_call(
    _a0_body,
    grid=(NB,),
    in_specs=[
        pl.BlockSpec((3, RB, DD), lambda i: (0, i, 0)),
        pl.BlockSpec((RR, DD, DD), lambda i: (0, 0, 0)),
    ],
    out_specs=[
        pl.BlockSpec((RB, DD), lambda i: (i, 0)),
        pl.BlockSpec((RR, RB, DD), lambda i: (0, i, 0)),
    ],
    out_shape=[
        jax.ShapeDtypeStruct((NP_, DD), jnp.float32),
        jax.ShapeDtypeStruct((RR, NP_, DD), jnp.float32),
    ],
)


@functools.cache
def _make_step(nres, with_hr):
    def body(*refs):
        h = refs[0][...]
        agg2 = refs[1]
        res = [refs[2 + i][...] for i in range(nres)]
        k = 2 + nres
        wih, whh, bih, bhh = refs[k], refs[k + 1], refs[k + 2], refs[k + 3]
        hnew_ref = refs[k + 4 + (1 if with_hr else 0)]
        agg = agg2[0] + agg2[1]
        xin = jnp.concatenate([agg] + res, axis=1) if nres else agg
        gi = lax.dot_general(
            xin, wih[...], (((1,), (1,)), ((), ())),
            preferred_element_type=jnp.float32,
        ) + bih[...]
        gh = lax.dot_general(
            h, whh[...], (((1,), (1,)), ((), ())),
            preferred_element_type=jnp.float32,
        ) + bhh[...]
        r = jax.nn.sigmoid(gi[:, :DD] + gh[:, :DD])
        z = jax.nn.sigmoid(gi[:, DD:2 * DD] + gh[:, DD:2 * DD])
        n = jnp.tanh(gi[:, 2 * DD:] + r * gh[:, 2 * DD:])
        hn = (1.0 - z) * n + z * h
        hnew_ref[...] = hn
        if with_hr:
            w = refs[k + 4]
            hr_ref = refs[k + 6]
            for rr in range(RR):
                hr_ref[rr] = jnp.dot(hn, w[rr], preferred_element_type=jnp.float32)

    in_dim = DD * (1 + nres)
    in_specs = [
        pl.BlockSpec((RB, DD), lambda i: (i, 0)),
        pl.BlockSpec((NC, RB, DD), lambda i: (0, i, 0)),
    ]
    for _ in range(nres):
        in_specs.append(pl.BlockSpec((RB, DD), lambda i: (i, 0)))
    in_specs += [
        pl.BlockSpec((3 * DD, in_dim), lambda i: (0, 0)),
        pl.BlockSpec((3 * DD, DD), lambda i: (0, 0)),
        pl.BlockSpec((1, 3 * DD), lambda i: (0, 0)),
        pl.BlockSpec((1, 3 * DD), lambda i: (0, 0)),
    ]
    out_specs = [pl.BlockSpec((RB, DD), lambda i: (i, 0))]
    out_shape = [jax.ShapeDtypeStruct((NP_, DD), jnp.float32)]
    if with_hr:
        in_specs.append(pl.BlockSpec((RR, DD, DD), lambda i: (0, 0, 0)))
        out_specs.append(pl.BlockSpec((RR, RB, DD), lambda i: (0, i, 0)))
        out_shape.append(jax.ShapeDtypeStruct((RR, NP_, DD), jnp.float32))
    return pl.pallas_call(
        body,
        grid=(NB,),
        in_specs=in_specs,
        out_specs=out_specs,
        out_shape=out_shape,
    )


def _c1_body(hf, h0, bt, clw, clb, crw, crb, g_ref):
    i = pl.program_id(0)
    hx = jnp.concatenate([hf[...], h0[...]], axis=1)
    a = jax.nn.sigmoid(
        lax.dot_general(hx, clw[...], (((1,), (1,)), ((), ())),
                        preferred_element_type=jnp.float32) + clb[...])
    b = jnp.tanh(
        lax.dot_general(hx, crw[...], (((1,), (1,)), ((), ())),
                        preferred_element_type=jnp.float32) + crb[...])
    node_out = a * b
    gids = lax.broadcasted_iota(jnp.int32, (RB, GG), 1)
    onehot = (bt[...] == gids).astype(jnp.float32)
    pool = lax.dot_general(onehot, node_out, (((0,), (0,)), ((), ())),
                           preferred_element_type=jnp.float32)

    @pl.when(i == 0)
    def _():
        g_ref[...] = jnp.zeros_like(g_ref)

    g_ref[...] += pool


_c1 = pl.pallas_call(
    _c1_body,
    grid=(NB,),
    in_specs=[
        pl.BlockSpec((RB, DD), lambda i: (i, 0)),
        pl.BlockSpec((RB, DD), lambda i: (i, 0)),
        pl.BlockSpec((RB, 1), lambda i: (i, 0)),
        pl.BlockSpec((DD, 2 * DD), lambda i: (0, 0)),
        pl.BlockSpec((1, DD), lambda i: (0, 0)),
        pl.BlockSpec((DD, 2 * DD), lambda i: (0, 0)),
        pl.BlockSpec((1, DD), lambda i: (0, 0)),
    ],
    out_specs=pl.BlockSpec((GG, DD), lambda i: (0, 0)),
    out_shape=jax.ShapeDtypeStruct((GG, DD), jnp.float32),
)


def _c2_body(g, pw, pb, out):
    out[0] = lax.dot_general(g[...], pw[0], (((1,), (1,)), ((), ())),
                             preferred_element_type=jnp.float32) + pb[0]


_c2 = pl.pallas_call(
    _c2_body,
    grid=(SEQ,),
    in_specs=[
        pl.BlockSpec((GG, DD), lambda s: (0, 0)),
        pl.BlockSpec((1, VP, DD), lambda s: (s, 0, 0)),
        pl.BlockSpec((1, 1, VP), lambda s: (s, 0, 0)),
    ],
    out_specs=pl.BlockSpec((1, GG, VP), lambda s: (s, 0, 0)),
    out_shape=jax.ShapeDtypeStruct((SEQ, GG, VP), jnp.float32),
)


def kernel(x, edge_index, node_depth, batch, edge_attr, params):
    i32 = jnp.int32
    f32 = jnp.float32
    x = x.astype(i32)
    src = edge_index[0].astype(i32)
    dst = edge_index[1].astype(i32)
    rel = edge_attr.astype(i32)

    table = jnp.concatenate(
        [params['type_emb'], params['attr_emb'], params['depth_emb']], axis=0)
    eidx = jnp.concatenate(
        [x[:, 0], 100 + x[:, 1], 1100 + node_depth.reshape(-1).astype(i32)])
    eidx3 = jnp.zeros((3, NP_), i32).at[:, :NN].set(eidx.reshape(3, NN))
    per_w = 3 * NP_ // NW
    eidx_w = (jnp.zeros((NW, ECH * 128), i32)
              .at[:, :per_w].set(eidx3.reshape(NW, per_w))
              .reshape(NW, ECH, 128))
    eout = _embed_gather(table, eidx_w)
    e3 = eout.reshape(NW, ECH * 128, DD)[:, :per_w].reshape(3, NP_, DD)

    gidx = rel * NP_ + src
    cap0 = NS * CH0 * 128

    def split_edges(flat, fill):
        a0 = flat[:cap0].reshape(NS, CH0, 128)
        a1 = flat[cap0:].reshape(NS, CH1, 128)
        pad = jnp.full((NS, CH0 - CH1, 128), fill, i32)
        return jnp.concatenate([a0, jnp.concatenate([a1, pad], 1)], 0)

    gidx_p = split_edges(jnp.zeros((NW * EW,), i32).at[:EE].set(gidx), 0)
    dst_p = split_edges(jnp.full((NW * EW,), NN, i32).at[:EE].set(dst), NN)
    zeros_np = jnp.zeros((NP_, DD), f32)

    steps = [l for l, T in enumerate(LAYER_T) for _ in range(T)]
    h, hr = _a0(e3, params['edge_w_0'])
    h0 = h
    states = [h0]
    for k, l in enumerate(steps):
        agg2 = _edge_gs(hr.reshape(RR * NP_, DD), gidx_p, dst_p, zeros_np)
        agg2 = agg2.reshape(NC, NP_, DD)
        res = [states[i] for i in RES_MAP.get(l, [])]
        last = k + 1 == len(steps)
        step_fn = _make_step(len(res), not last)
        args = [h, agg2] + res + [
            params['gru_wih_%d' % l],
            params['gru_whh_%d' % l],
            params['gru_bih_%d' % l].reshape(1, 3 * DD),
            params['gru_bhh_%d' % l].reshape(1, 3 * DD),
        ]
        if last:
            (h,) = step_fn(*args)
        else:
            nl = steps[k + 1]
            h, hr = step_fn(*(args + [params['edge_w_%d' % nl]]))
        if last or steps[k + 1] != l:
            states.append(h)

    batch_p = jnp.full((NP_, 1), GG, i32).at[:NN, 0].set(batch.astype(i32))
    g = _c1(states[-1], h0, batch_p,
            params['cl_w'], params['cl_b'].reshape(1, DD),
            params['cr_w'], params['cr_b'].reshape(1, DD))
    pw = jnp.zeros((SEQ, VP, DD), f32).at[:, :VV].set(params['pred_w'])
    pb = jnp.zeros((SEQ, 1, VP), f32).at[:, 0, :VV].set(params['pred_b'])
    preds = _c2(g, pw, pb)
    return preds[:, :, :VV]


# speedup vs baseline: 1.5461x; 1.0068x over previous
import functools

import jax
import jax.numpy as jnp
from jax import lax
from jax.experimental import pallas as pl
from jax.experimental.pallas import tpu as pltpu
from jax.experimental.pallas import tpu_sc as plsc

NN = 10000
EE = 320000
DD = 128
RR = 4
NP_ = 10240
NC, NS = 2, 16
NW = NC * NS
CH = 79
CH0, CH1 = 99, 59
EW = CH * 128
NPA = NP_
RPT = NPA // NS
ECH = 8
GG = 128
SEQ = 5
VV = 5000
VP = 5120
RB = 256
NB = NP_ // RB
LAYER_T = [2, 2, 1, 2, 1]
RES_MAP = {2: [0], 4: [0, 2]}


def _embed_gather_body(table, eidx, out, idx_v, buf, sem):
    c = lax.axis_index("c")
    s = lax.axis_index("s")
    wid = s * NC + c
    pltpu.sync_copy(eidx.at[wid], idx_v)
    for j in range(ECH):
        pltpu.async_copy(table.at[idx_v.at[j]], buf, sem).wait()
        pltpu.sync_copy(buf, out.at[pl.ds(wid * (ECH * 128) + j * 128, 128)])


@functools.cache
def _sc_mesh():
    return plsc.VectorSubcoreMesh(core_axis_name="c", subcore_axis_name="s")


@functools.cache
def _embed_gather_kernel():
    return pl.kernel(
        _embed_gather_body,
        out_type=jax.ShapeDtypeStruct((NW * ECH * 128, DD), jnp.float32),
        mesh=_sc_mesh(),
        scratch_types=[
            pltpu.VMEM((ECH, 128), jnp.int32),
            pltpu.VMEM((128, DD), jnp.float32),
            pltpu.SemaphoreType.DMA,
        ],
    )


def _embed_gather(table, eidx_w):
    return _embed_gather_kernel()(table, eidx_w)


def _unpack_row(packed, j, stage, b):
    half = (j % 2) * 64
    for k in range(4):
        w = packed[j // 2, pl.ds(half + k * 16, 16)]
        stage[b, pl.ds(k * 16, 16)] = (w & 0xFFFF).astype(jnp.int32)
        stage[b, pl.ds(64 + k * 16, 16)] = (w >> 16).astype(jnp.int32)


def _edge_gs_body(hr, gidx, dstl, out, shared, gbuf, gidx_v, dst_v, gsem):
    c = lax.axis_index("c")
    s = lax.axis_index("s")
    widx = c * NS + s

    def zrow(i, carry):
        for k in range(8):
            gbuf[i, pl.ds(k * 16, 16)] = jnp.zeros((16,), jnp.float32)
        return carry

    lax.fori_loop(0, 128, zrow, 0)
    for t in range(RPT // 128):
        pltpu.sync_copy(gbuf, shared.at[pl.ds(s * RPT + t * 128, 128)])
    pltpu.sync_copy(gidx.at[widx], gidx_v)
    pltpu.sync_copy(dstl.at[widx], dst_v)
    plsc.subcore_barrier()

    def body(j, carry):
        pltpu.async_copy(hr.at[gidx_v.at[j]], gbuf, gsem).wait()
        pltpu.sync_copy(gbuf, shared.at[dst_v.at[j]], add=True)
        return carry

    lax.fori_loop(0, jnp.where(c == 0, CH0, CH1), body, 0)
    plsc.subcore_barrier()
    for t in range(RPT // 128):
        pltpu.sync_copy(shared.at[pl.ds(s * RPT + t * 128, 128)], gbuf)
        pltpu.sync_copy(gbuf, out.at[pl.ds(c * NP_ + s * RPT + t * 128, 128)])


@functools.cache
def _edge_gs_kernel():
    return pl.kernel(
        _edge_gs_body,
        out_type=jax.ShapeDtypeStruct((NC * NP_, DD), jnp.float32),
        mesh=_sc_mesh(),
        scratch_types=[
            pltpu.VMEM_SHARED((NPA, DD), jnp.float32),
            pltpu.VMEM((128, DD), jnp.float32),
            pltpu.VMEM((CH0, 128), jnp.int32),
            pltpu.VMEM((CH0, 128), jnp.int32),
            pltpu.SemaphoreType.DMA,
        ],
    )


def _edge_gs(hr_flat, gidx_p, dst_p):
    return _edge_gs_kernel()(hr_flat, gidx_p, dst_p)


def _a0_body(e3, w, h0_ref, hr_ref):
    h0 = e3[0] + e3[1] + e3[2]
    h0_ref[...] = h0
    for r in range(RR):
        hr_ref[r] = jnp.dot(h0, w[r], preferred_element_type=jnp.float32)


_a0 = pl.pallas_call(
    _a0_body,
    grid=(NB,),
    in_specs=[
        pl.BlockSpec((3, RB, DD), lambda i: (0, i, 0)),
        pl.BlockSpec((RR, DD, DD), lambda i: (0, 0, 0)),
    ],
    out_specs=[
        pl.BlockSpec((RB, DD), lambda i: (i, 0)),
        pl.BlockSpec((RR, RB, DD), lambda i: (0, i, 0)),
    ],
    out_shape=[
        jax.ShapeDtypeStruct((NP_, DD), jnp.float32),
        jax.ShapeDtypeStruct((RR, NP_, DD), jnp.float32),
    ],
)


@functools.cache
def _make_step(nres, with_hr):
    def body(*refs):
        h = refs[0][...]
        agg2 = refs[1]
        res = [refs[2 + i][...] for i in range(nres)]
        k = 2 + nres
        wih, whh, bih, bhh = refs[k], refs[k + 1], refs[k + 2], refs[k + 3]
        hnew_ref = refs[k + 4 + (1 if with_hr else 0)]
        agg = agg2[0] + agg2[1]
        xin = jnp.concatenate([agg] + res, axis=1) if nres else agg
        gi = lax.dot_general(
            xin, wih[...], (((1,), (1,)), ((), ())),
            preferred_element_type=jnp.float32,
        ) + bih[...]
        gh = lax.dot_general(
            h, whh[...], (((1,), (1,)), ((), ())),
            preferred_element_type=jnp.float32,
        ) + bhh[...]
        r = jax.nn.sigmoid(gi[:, :DD] + gh[:, :DD])
        z = jax.nn.sigmoid(gi[:, DD:2 * DD] + gh[:, DD:2 * DD])
        n = jnp.tanh(gi[:, 2 * DD:] + r * gh[:, 2 * DD:])
        hn = (1.0 - z) * n + z * h
        hnew_ref[...] = hn
        if with_hr:
            w = refs[k + 4]
            hr_ref = refs[k + 6]
            for rr in range(RR):
                hr_ref[rr] = jnp.dot(hn, w[rr], preferred_element_type=jnp.float32)

    in_dim = DD * (1 + nres)
    in_specs = [
        pl.BlockSpec((RB, DD), lambda i: (i, 0)),
        pl.BlockSpec((NC, RB, DD), lambda i: (0, i, 0)),
    ]
    for _ in range(nres):
        in_specs.append(pl.BlockSpec((RB, DD), lambda i: (i, 0)))
    in_specs += [
        pl.BlockSpec((3 * DD, in_dim), lambda i: (0, 0)),
        pl.BlockSpec((3 * DD, DD), lambda i: (0, 0)),
        pl.BlockSpec((1, 3 * DD), lambda i: (0, 0)),
        pl.BlockSpec((1, 3 * DD), lambda i: (0, 0)),
    ]
    out_specs = [pl.BlockSpec((RB, DD), lambda i: (i, 0))]
    out_shape = [jax.ShapeDtypeStruct((NP_, DD), jnp.float32)]
    if with_hr:
        in_specs.append(pl.BlockSpec((RR, DD, DD), lambda i: (0, 0, 0)))
        out_specs.append(pl.BlockSpec((RR, RB, DD), lambda i: (0, i, 0)))
        out_shape.append(jax.ShapeDtypeStruct((RR, NP_, DD), jnp.float32))
    return pl.pallas_call(
        body,
        grid=(NB,),
        in_specs=in_specs,
        out_specs=out_specs,
        out_shape=out_shape,
    )


def _c1_body(hf, h0, bt, clw, clb, crw, crb, g_ref):
    i = pl.program_id(0)
    hx = jnp.concatenate([hf[...], h0[...]], axis=1)
    a = jax.nn.sigmoid(
        lax.dot_general(hx, clw[...], (((1,), (1,)), ((), ())),
                        preferred_element_type=jnp.float32) + clb[...])
    b = jnp.tanh(
        lax.dot_general(hx, crw[...], (((1,), (1,)), ((), ())),
                        preferred_element_type=jnp.float32) + crb[...])
    node_out = a * b
    gids = lax.broadcasted_iota(jnp.int32, (RB, GG), 1)
    onehot = (bt[...] == gids).astype(jnp.float32)
    pool = lax.dot_general(onehot, node_out, (((0,), (0,)), ((), ())),
                           preferred_element_type=jnp.float32)

    @pl.when(i == 0)
    def _():
        g_ref[...] = jnp.zeros_like(g_ref)

    g_ref[...] += pool


_c1 = pl.pallas_call(
    _c1_body,
    grid=(NB,),
    in_specs=[
        pl.BlockSpec((RB, DD), lambda i: (i, 0)),
        pl.BlockSpec((RB, DD), lambda i: (i, 0)),
        pl.BlockSpec((RB, 1), lambda i: (i, 0)),
        pl.BlockSpec((DD, 2 * DD), lambda i: (0, 0)),
        pl.BlockSpec((1, DD), lambda i: (0, 0)),
        pl.BlockSpec((DD, 2 * DD), lambda i: (0, 0)),
        pl.BlockSpec((1, DD), lambda i: (0, 0)),
    ],
    out_specs=pl.BlockSpec((GG, DD), lambda i: (0, 0)),
    out_shape=jax.ShapeDtypeStruct((GG, DD), jnp.float32),
)


def _c2_body(g, pw, pb, out):
    out[0] = lax.dot_general(g[...], pw[0], (((1,), (1,)), ((), ())),
                             preferred_element_type=jnp.float32) + pb[0]


_c2 = pl.pallas_call(
    _c2_body,
    grid=(SEQ,),
    in_specs=[
        pl.BlockSpec((GG, DD), lambda s: (0, 0)),
        pl.BlockSpec((1, VP, DD), lambda s: (s, 0, 0)),
        pl.BlockSpec((1, 1, VP), lambda s: (s, 0, 0)),
    ],
    out_specs=pl.BlockSpec((1, GG, VP), lambda s: (s, 0, 0)),
    out_shape=jax.ShapeDtypeStruct((SEQ, GG, VP), jnp.float32),
)


def kernel(x, edge_index, node_depth, batch, edge_attr, params):
    i32 = jnp.int32
    f32 = jnp.float32
    x = x.astype(i32)
    src = edge_index[0].astype(i32)
    dst = edge_index[1].astype(i32)
    rel = edge_attr.astype(i32)

    table = jnp.concatenate(
        [params['type_emb'], params['attr_emb'], params['depth_emb']], axis=0)
    eidx = jnp.concatenate(
        [x[:, 0], 100 + x[:, 1], 1100 + node_depth.reshape(-1).astype(i32)])
    eidx3 = jnp.zeros((3, NP_), i32).at[:, :NN].set(eidx.reshape(3, NN))
    per_w = 3 * NP_ // NW
    eidx_w = (jnp.zeros((NW, ECH * 128), i32)
              .at[:, :per_w].set(eidx3.reshape(NW, per_w))
              .reshape(NW, ECH, 128))
    eout = _embed_gather(table, eidx_w)
    e3 = eout.reshape(NW, ECH * 128, DD)[:, :per_w].reshape(3, NP_, DD)

    gidx = rel * NP_ + src
    cap0 = NS * CH0 * 128

    def split_edges(flat, fill):
        a0 = flat[:cap0].reshape(NS, CH0, 128)
        a1 = flat[cap0:].reshape(NS, CH1, 128)
        pad = jnp.full((NS, CH0 - CH1, 128), fill, i32)
        return jnp.concatenate([a0, jnp.concatenate([a1, pad], 1)], 0)

    gidx_p = split_edges(jnp.zeros((NW * EW,), i32).at[:EE].set(gidx), 0)
    dst_p = split_edges(jnp.full((NW * EW,), NN, i32).at[:EE].set(dst), NN)

    steps = [l for l, T in enumerate(LAYER_T) for _ in range(T)]
    h, hr = _a0(e3, params['edge_w_0'])
    h0 = h
    states = [h0]
    for k, l in enumerate(steps):
        agg2 = _edge_gs(hr.reshape(RR * NP_, DD), gidx_p, dst_p)
        agg2 = agg2.reshape(NC, NP_, DD)
        res = [states[i] for i in RES_MAP.get(l, [])]
        last = k + 1 == len(steps)
        step_fn = _make_step(len(res), not last)
        args = [h, agg2] + res + [
            params['gru_wih_%d' % l],
            params['gru_whh_%d' % l],
            params['gru_bih_%d' % l].reshape(1, 3 * DD),
            params['gru_bhh_%d' % l].reshape(1, 3 * DD),
        ]
        if last:
            (h,) = step_fn(*args)
        else:
            nl = steps[k + 1]
            h, hr = step_fn(*(args + [params['edge_w_%d' % nl]]))
        if last or steps[k + 1] != l:
            states.append(h)

    batch_p = jnp.full((NP_, 1), GG, i32).at[:NN, 0].set(batch.astype(i32))
    g = _c1(states[-1], h0, batch_p,
            params['cl_w'], params['cl_b'].reshape(1, DD),
            params['cr_w'], params['cr_b'].reshape(1, DD))
    pw = jnp.zeros((SEQ, VP, DD), f32).at[:, :VV].set(params['pred_w'])
    pb = jnp.zeros((SEQ, 1, VP), f32).at[:, 0, :VV].set(params['pred_b'])
    preds = _c2(g, pw, pb)
    return preds[:, :, :VV]


# speedup vs baseline: 1.5544x; 1.0054x over previous
import functools

import jax
import jax.numpy as jnp
from jax import lax
from jax.experimental import pallas as pl
from jax.experimental.pallas import tpu as pltpu
from jax.experimental.pallas import tpu_sc as plsc

NN = 10000
EE = 320000
DD = 128
RR = 4
NP_ = 10240
NC, NS = 2, 16
NW = NC * NS
CH = 79
CH0, CH1 = 111, 47
EW = CH * 128
NPA = NP_
RPT = NPA // NS
ECH = 8
GG = 128
SEQ = 5
VV = 5000
VP = 5120
RB = 256
NB = NP_ // RB
LAYER_T = [2, 2, 1, 2, 1]
RES_MAP = {2: [0], 4: [0, 2]}


def _embed_gather_body(table, eidx, out, idx_v, buf, sem):
    c = lax.axis_index("c")
    s = lax.axis_index("s")
    wid = s * NC + c
    pltpu.sync_copy(eidx.at[wid], idx_v)
    for j in range(ECH):
        pltpu.async_copy(table.at[idx_v.at[j]], buf, sem).wait()
        pltpu.sync_copy(buf, out.at[pl.ds(wid * (ECH * 128) + j * 128, 128)])


@functools.cache
def _sc_mesh():
    return plsc.VectorSubcoreMesh(core_axis_name="c", subcore_axis_name="s")


@functools.cache
def _embed_gather_kernel():
    return pl.kernel(
        _embed_gather_body,
        out_type=jax.ShapeDtypeStruct((NW * ECH * 128, DD), jnp.float32),
        mesh=_sc_mesh(),
        scratch_types=[
            pltpu.VMEM((ECH, 128), jnp.int32),
            pltpu.VMEM((128, DD), jnp.float32),
            pltpu.SemaphoreType.DMA,
        ],
    )


def _embed_gather(table, eidx_w):
    return _embed_gather_kernel()(table, eidx_w)


def _unpack_row(packed, j, stage, b):
    half = (j % 2) * 64
    for k in range(4):
        w = packed[j // 2, pl.ds(half + k * 16, 16)]
        stage[b, pl.ds(k * 16, 16)] = (w & 0xFFFF).astype(jnp.int32)
        stage[b, pl.ds(64 + k * 16, 16)] = (w >> 16).astype(jnp.int32)


def _edge_gs_body(hr, gidx, dstl, out, shared, gbuf, gidx_v, dst_v, gsem):
    c = lax.axis_index("c")
    s = lax.axis_index("s")
    widx = c * NS + s

    def zrow(i, carry):
        for k in range(8):
            gbuf[i, pl.ds(k * 16, 16)] = jnp.zeros((16,), jnp.float32)
        return carry

    lax.fori_loop(0, 128, zrow, 0)
    for t in range(RPT // 128):
        pltpu.sync_copy(gbuf, shared.at[pl.ds(s * RPT + t * 128, 128)])
    pltpu.sync_copy(gidx.at[widx], gidx_v)
    pltpu.sync_copy(dstl.at[widx], dst_v)
    plsc.subcore_barrier()

    def body(j, carry):
        pltpu.async_copy(hr.at[gidx_v.at[j]], gbuf, gsem).wait()
        pltpu.sync_copy(gbuf, shared.at[dst_v.at[j]], add=True)
        return carry

    lax.fori_loop(0, jnp.where(c == 0, CH0, CH1), body, 0)
    plsc.subcore_barrier()
    for t in range(RPT // 128):
        pltpu.sync_copy(shared.at[pl.ds(s * RPT + t * 128, 128)], gbuf)
        pltpu.sync_copy(gbuf, out.at[pl.ds(c * NP_ + s * RPT + t * 128, 128)])


@functools.cache
def _edge_gs_kernel():
    return pl.kernel(
        _edge_gs_body,
        out_type=jax.ShapeDtypeStruct((NC * NP_, DD), jnp.float32),
        mesh=_sc_mesh(),
        scratch_types=[
            pltpu.VMEM_SHARED((NPA, DD), jnp.float32),
            pltpu.VMEM((128, DD), jnp.float32),
            pltpu.VMEM((CH0, 128), jnp.int32),
            pltpu.VMEM((CH0, 128), jnp.int32),
            pltpu.SemaphoreType.DMA,
        ],
    )


def _edge_gs(hr_flat, gidx_p, dst_p):
    return _edge_gs_kernel()(hr_flat, gidx_p, dst_p)


def _a0_body(e3, w, h0_ref, hr_ref):
    h0 = e3[0] + e3[1] + e3[2]
    h0_ref[...] = h0
    for r in range(RR):
        hr_ref[r] = jnp.dot(h0, w[r], preferred_element_type=jnp.float32)


_a0 = pl.pallas_call(
    _a0_body,
    grid=(NB,),
    in_specs=[
        pl.BlockSpec((3, RB, DD), lambda i: (0, i, 0)),
        pl.BlockSpec((RR, DD, DD), lambda i: (0, 0, 0)),
    ],
    out_specs=[
        pl.BlockSpec((RB, DD), lambda i: (i, 0)),
        pl.BlockSpec((RR, RB, DD), lambda i: (0, i, 0)),
    ],
    out_shape=[
        jax.ShapeDtypeStruct((NP_, DD), jnp.float32),
        jax.ShapeDtypeStruct((RR, NP_, DD), jnp.float32),
    ],
)


@functools.cache
def _make_step(nres, with_hr):
    def body(*refs):
        h = refs[0][...]
        agg2 = refs[1]
        res = [refs[2 + i][...] for i in range(nres)]
        k = 2 + nres
        wih, whh, bih, bhh = refs[k], refs[k + 1], refs[k + 2], refs[k + 3]
        hnew_ref = refs[k + 4 + (1 if with_hr else 0)]
        agg = agg2[0] + agg2[1]
        xin = jnp.concatenate([agg] + res, axis=1) if nres else agg
        gi = lax.dot_general(
            xin, wih[...], (((1,), (1,)), ((), ())),
            preferred_element_type=jnp.float32,
        ) + bih[...]
        gh = lax.dot_general(
            h, whh[...], (((1,), (1,)), ((), ())),
            preferred_element_type=jnp.float32,
        ) + bhh[...]
        r = jax.nn.sigmoid(gi[:, :DD] + gh[:, :DD])
        z = jax.nn.sigmoid(gi[:, DD:2 * DD] + gh[:, DD:2 * DD])
        n = jnp.tanh(gi[:, 2 * DD:] + r * gh[:, 2 * DD:])
        hn = (1.0 - z) * n + z * h
        hnew_ref[...] = hn
        if with_hr:
            w = refs[k + 4]
            hr_ref = refs[k + 6]
            for rr in range(RR):
                hr_ref[rr] = jnp.dot(hn, w[rr], preferred_element_type=jnp.float32)

    in_dim = DD * (1 + nres)
    in_specs = [
        pl.BlockSpec((RB, DD), lambda i: (i, 0)),
        pl.BlockSpec((NC, RB, DD), lambda i: (0, i, 0)),
    ]
    for _ in range(nres):
        in_specs.append(pl.BlockSpec((RB, DD), lambda i: (i, 0)))
    in_specs += [
        pl.BlockSpec((3 * DD, in_dim), lambda i: (0, 0)),
        pl.BlockSpec((3 * DD, DD), lambda i: (0, 0)),
        pl.BlockSpec((1, 3 * DD), lambda i: (0, 0)),
        pl.BlockSpec((1, 3 * DD), lambda i: (0, 0)),
    ]
    out_specs = [pl.BlockSpec((RB, DD), lambda i: (i, 0))]
    out_shape = [jax.ShapeDtypeStruct((NP_, DD), jnp.float32)]
    if with_hr:
        in_specs.append(pl.BlockSpec((RR, DD, DD), lambda i: (0, 0, 0)))
        out_specs.append(pl.BlockSpec((RR, RB, DD), lambda i: (0, i, 0)))
        out_shape.append(jax.ShapeDtypeStruct((RR, NP_, DD), jnp.float32))
    return pl.pallas_call(
        body,
        grid=(NB,),
        in_specs=in_specs,
        out_specs=out_specs,
        out_shape=out_shape,
    )


def _c1_body(hf, h0, bt, clw, clb, crw, crb, g_ref):
    i = pl.program_id(0)
    hx = jnp.concatenate([hf[...], h0[...]], axis=1)
    a = jax.nn.sigmoid(
        lax.dot_general(hx, clw[...], (((1,), (1,)), ((), ())),
                        preferred_element_type=jnp.float32) + clb[...])
    b = jnp.tanh(
        lax.dot_general(hx, crw[...], (((1,), (1,)), ((), ())),
                        preferred_element_type=jnp.float32) + crb[...])
    node_out = a * b
    gids = lax.broadcasted_iota(jnp.int32, (RB, GG), 1)
    onehot = (bt[...] == gids).astype(jnp.float32)
    pool = lax.dot_general(onehot, node_out, (((0,), (0,)), ((), ())),
                           preferred_element_type=jnp.float32)

    @pl.when(i == 0)
    def _():
        g_ref[...] = jnp.zeros_like(g_ref)

    g_ref[...] += pool


_c1 = pl.pallas_call(
    _c1_body,
    grid=(NB,),
    in_specs=[
        pl.BlockSpec((RB, DD), lambda i: (i, 0)),
        pl.BlockSpec((RB, DD), lambda i: (i, 0)),
        pl.BlockSpec((RB, 1), lambda i: (i, 0)),
        pl.BlockSpec((DD, 2 * DD), lambda i: (0, 0)),
        pl.BlockSpec((1, DD), lambda i: (0, 0)),
        pl.BlockSpec((DD, 2 * DD), lambda i: (0, 0)),
        pl.BlockSpec((1, DD), lambda i: (0, 0)),
    ],
    out_specs=pl.BlockSpec((GG, DD), lambda i: (0, 0)),
    out_shape=jax.ShapeDtypeStruct((GG, DD), jnp.float32),
)


def _c2_body(g, pw, pb, out):
    out[0] = lax.dot_general(g[...], pw[0], (((1,), (1,)), ((), ())),
                             preferred_element_type=jnp.float32) + pb[0]


_c2 = pl.pallas_call(
    _c2_body,
    grid=(SEQ,),
    in_specs=[
        pl.BlockSpec((GG, DD), lambda s: (0, 0)),
        pl.BlockSpec((1, VP, DD), lambda s: (s, 0, 0)),
        pl.BlockSpec((1, 1, VP), lambda s: (s, 0, 0)),
    ],
    out_specs=pl.BlockSpec((1, GG, VP), lambda s: (s, 0, 0)),
    out_shape=jax.ShapeDtypeStruct((SEQ, GG, VP), jnp.float32),
)


def kernel(x, edge_index, node_depth, batch, edge_attr, params):
    i32 = jnp.int32
    f32 = jnp.float32
    x = x.astype(i32)
    src = edge_index[0].astype(i32)
    dst = edge_index[1].astype(i32)
    rel = edge_attr.astype(i32)

    table = jnp.concatenate(
        [params['type_emb'], params['attr_emb'], params['depth_emb']], axis=0)
    eidx = jnp.concatenate(
        [x[:, 0], 100 + x[:, 1], 1100 + node_depth.reshape(-1).astype(i32)])
    eidx3 = jnp.zeros((3, NP_), i32).at[:, :NN].set(eidx.reshape(3, NN))
    per_w = 3 * NP_ // NW
    eidx_w = (jnp.zeros((NW, ECH * 128), i32)
              .at[:, :per_w].set(eidx3.reshape(NW, per_w))
              .reshape(NW, ECH, 128))
    eout = _embed_gather(table, eidx_w)
    e3 = eout.reshape(NW, ECH * 128, DD)[:, :per_w].reshape(3, NP_, DD)

    gidx = rel * NP_ + src
    cap0 = NS * CH0 * 128

    def split_edges(flat, fill):
        a0 = flat[:cap0].reshape(NS, CH0, 128)
        a1 = flat[cap0:].reshape(NS, CH1, 128)
        pad = jnp.full((NS, CH0 - CH1, 128), fill, i32)
        return jnp.concatenate([a0, jnp.concatenate([a1, pad], 1)], 0)

    gidx_p = split_edges(jnp.zeros((NW * EW,), i32).at[:EE].set(gidx), 0)
    dst_p = split_edges(jnp.full((NW * EW,), NN, i32).at[:EE].set(dst), NN)

    steps = [l for l, T in enumerate(LAYER_T) for _ in range(T)]
    h, hr = _a0(e3, params['edge_w_0'])
    h0 = h
    states = [h0]
    for k, l in enumerate(steps):
        agg2 = _edge_gs(hr.reshape(RR * NP_, DD), gidx_p, dst_p)
        agg2 = agg2.reshape(NC, NP_, DD)
        res = [states[i] for i in RES_MAP.get(l, [])]
        last = k + 1 == len(steps)
        step_fn = _make_step(len(res), not last)
        args = [h, agg2] + res + [
            params['gru_wih_%d' % l],
            params['gru_whh_%d' % l],
            params['gru_bih_%d' % l].reshape(1, 3 * DD),
            params['gru_bhh_%d' % l].reshape(1, 3 * DD),
        ]
        if last:
            (h,) = step_fn(*args)
        else:
            nl = steps[k + 1]
            h, hr = step_fn(*(args + [params['edge_w_%d' % nl]]))
        if last or steps[k + 1] != l:
            states.append(h)

    batch_p = jnp.full((NP_, 1), GG, i32).at[:NN, 0].set(batch.astype(i32))
    g = _c1(states[-1], h0, batch_p,
            params['cl_w'], params['cl_b'].reshape(1, DD),
            params['cr_w'], params['cr_b'].reshape(1, DD))
    pw = jnp.zeros((SEQ, VP, DD), f32).at[:, :VV].set(params['pred_w'])
    pb = jnp.zeros((SEQ, 1, VP), f32).at[:, 0, :VV].set(params['pred_b'])
    preds = _c2(g, pw, pb)
    return preds[:, :, :VV]
